# Initial kernel scaffold; baseline (speedup 1.0000x reference)
import jax
import jax.numpy as jnp
from jax.experimental import pallas as pl


def kernel(early, middle, deep):
    raise NotImplementedError("write your pallas kernel here")


# speedup vs baseline: 44.0495x; 44.0495x over previous
import jax
import jax.numpy as jnp
import numpy as np
from jax.experimental import pallas as pl

_SIGN = np.uint32(0x80000000)
_LOWMASK = np.uint32(0x7FFFFFFF)

_RANK = 100000

def _to_key(x_f32):
    b = jax.lax.bitcast_convert_type(x_f32, jnp.uint32)
    return jnp.where(b >= _SIGN, ~b, b | _SIGN)


def _from_key(k_u32):
    b = jnp.where(k_u32 >= _SIGN, k_u32 & _LOWMASK, ~k_u32)
    return jax.lax.bitcast_convert_type(b, jnp.float32)


def _select_and_mask(e_ref, m_ref, d_ref, oe_ref, om_ref, od_ref):
    oe_ref[...] = jax.lax.bitcast_convert_type(_to_key(e_ref[...]), jnp.float32)
    om_ref[...] = jax.lax.bitcast_convert_type(_to_key(m_ref[...]), jnp.float32)
    od_ref[...] = jax.lax.bitcast_convert_type(_to_key(d_ref[...]), jnp.float32)

    def keys(ref):
        return jax.lax.bitcast_convert_type(ref[...], jnp.uint32)

    def bit_step(i, carry):
        prefix, decided, r = carry
        bitv = jnp.uint32(1) << (jnp.uint32(31) - i.astype(jnp.uint32))
        m2 = decided | bitv
        p2 = prefix | bitv
        c = (
            jnp.sum((keys(oe_ref) & m2) == p2, dtype=jnp.int32)
            + jnp.sum((keys(om_ref) & m2) == p2, dtype=jnp.int32)
            + jnp.sum((keys(od_ref) & m2) == p2, dtype=jnp.int32)
        )
        take = r < c
        prefix = jnp.where(take, p2, prefix)
        r = jnp.where(take, r, r - c)
        return prefix, m2, r

    prefix, _, _ = jax.lax.fori_loop(
        0, 32, bit_step,
        (jnp.uint32(0), jnp.uint32(0), jnp.int32(_RANK)),
    )
    thresh = _from_key(prefix)

    for ref in (oe_ref, om_ref, od_ref):
        x = _from_key(jax.lax.bitcast_convert_type(ref[...], jnp.uint32))
        ref[...] = jnp.where(x < thresh, jnp.float32(0.0), x)


def kernel(early, middle, deep):
    shp = early.shape
    e2 = early.reshape(1024, 1024)
    m2 = middle.reshape(1024, 1024)
    d2 = deep.reshape(1024, 1024)
    out = jax.ShapeDtypeStruct((1024, 1024), jnp.float32)
    oe, om, od = pl.pallas_call(
        _select_and_mask,
        out_shape=(out, out, out),
    )(e2, m2, d2)
    return (oe.reshape(shp), om.reshape(shp), od.reshape(shp))
